# Initial kernel scaffold; baseline (speedup 1.0000x reference)
import jax
import jax.numpy as jnp
from jax.experimental import pallas as pl


def kernel(vertices, feature_map):
    raise NotImplementedError("write your pallas kernel here")


# speedup vs baseline: 32.2526x; 32.2526x over previous
import functools

import jax
import jax.numpy as jnp
import numpy as np
from jax import lax
from jax.experimental import pallas as pl
from jax.experimental.pallas import tpu as pltpu
from jax.experimental.pallas import tpu_sc as plsc

POOLING_RATE = 4
NEIGHBOR_NUM = 16
BS = 8
N = 2048
C = 512
PN = N // POOLING_RATE
K1 = NEIGHBOR_NUM + 1

_SAMPLE_IDX = np.asarray(
    jax.random.permutation(jax.random.key(42), N)[:PN], dtype=np.int32)


def _knn_body(vs_ref, vt_ref, out_ref):
    vs = vs_ref[0]
    vt = vt_ref[0]
    inner = lax.dot_general(
        vs.astype(jnp.bfloat16), vt.astype(jnp.bfloat16),
        (((1,), (0,)), ((), ())), preferred_element_type=jnp.float32)
    q = vt[0:1, :] * vt[0:1, :] + vt[1:2, :] * vt[1:2, :] + vt[2:3, :] * vt[2:3, :]
    qs = vs[:, 0:1] * vs[:, 0:1] + vs[:, 1:2] * vs[:, 1:2] + vs[:, 2:3] * vs[:, 2:3]
    d = inner * (-2.0) + q + qs

    col = lax.broadcasted_iota(jnp.int32, (PN, N), 1)
    col16 = lax.broadcasted_iota(jnp.int32, (PN, NEIGHBOR_NUM), 1)
    base = pl.program_id(0) * N
    out = jnp.zeros((PN, NEIGHBOR_NUM), jnp.int32)
    for t in range(K1):
        m = jnp.min(d, axis=1, keepdims=True)
        idx = jnp.min(jnp.where(d == m, col, N), axis=1, keepdims=True)
        if t > 0:
            out = jnp.where(col16 == (t - 1), idx + base, out)
        if t < K1 - 1:
            d = jnp.where(col == idx, jnp.inf, d)
    out_ref[0] = out


def _knn_indices(vs, vt):
    return pl.pallas_call(
        _knn_body,
        grid=(BS,),
        in_specs=[
            pl.BlockSpec((1, PN, 3), lambda b: (b, 0, 0)),
            pl.BlockSpec((1, 3, N), lambda b: (b, 0, 0)),
        ],
        out_specs=pl.BlockSpec((1, PN, NEIGHBOR_NUM), lambda b: (b, 0, 0)),
        out_shape=jax.ShapeDtypeStruct((BS, PN, NEIGHBOR_NUM), jnp.int32),
    )(vs, vt)


_ROWS = BS * PN
_NW = 32
_RPW = _ROWS // _NW
_L = 16


def _pool_sc(f_hbm, idx_hbm):
    mesh = plsc.VectorSubcoreMesh(core_axis_name="c", subcore_axis_name="s")

    @functools.partial(
        pl.kernel,
        mesh=mesh,
        out_type=jax.ShapeDtypeStruct((_ROWS, C), jnp.float32),
        scratch_types=[
            pltpu.VMEM((_RPW, NEIGHBOR_NUM), jnp.int32),
            pltpu.VMEM((NEIGHBOR_NUM, C), jnp.float32),
            pltpu.VMEM((_RPW, C), jnp.float32),
            pltpu.SemaphoreType.DMA,
        ],
    )
    def k(f_ref, idx_ref, out_ref, idx_v, rows_v, out_v, sem):
        wid = lax.axis_index("s") * 2 + lax.axis_index("c")
        base = wid * _RPW
        pltpu.sync_copy(idx_ref.at[pl.ds(base, _RPW)], idx_v)

        def body(r, carry):
            pltpu.async_copy(f_ref.at[idx_v.at[r]], rows_v, sem).wait()
            for c0 in range(0, C, _L):
                acc = rows_v[0, pl.ds(c0, _L)]
                for j in range(1, NEIGHBOR_NUM):
                    acc = jnp.maximum(acc, rows_v[j, pl.ds(c0, _L)])
                out_v[r, pl.ds(c0, _L)] = acc
            return carry

        lax.fori_loop(0, _RPW, body, 0)
        pltpu.sync_copy(out_v, out_ref.at[pl.ds(base, _RPW)])

    return k(f_hbm, idx_hbm)


def kernel(vertices, feature_map):
    sidx = jnp.asarray(_SAMPLE_IDX)
    vs = jnp.take(vertices, sidx, axis=1)
    vt = jnp.transpose(vertices, (0, 2, 1))
    nbr = _knn_indices(vs, vt)
    f_flat = feature_map.reshape(BS * N, C)
    idx_flat = nbr.reshape(_ROWS, NEIGHBOR_NUM)
    pooled = _pool_sc(f_flat, idx_flat)
    return vs, pooled.reshape(BS, PN, C)


# speedup vs baseline: 45.3107x; 1.4049x over previous
import functools

import jax
import jax.numpy as jnp
import numpy as np
from jax import lax
from jax.experimental import pallas as pl
from jax.experimental.pallas import tpu as pltpu
from jax.experimental.pallas import tpu_sc as plsc

POOLING_RATE = 4
NEIGHBOR_NUM = 16
BS = 8
N = 2048
C = 512
PN = N // POOLING_RATE
K1 = NEIGHBOR_NUM + 1

_SAMPLE_IDX = np.asarray(
    jax.random.permutation(jax.random.key(42), N)[:PN], dtype=np.int32)


def _knn_body(vs_ref, vt_ref, out_ref):
    vs = vs_ref[0]
    vt = vt_ref[0]
    inner = lax.dot_general(
        vs.astype(jnp.bfloat16), vt.astype(jnp.bfloat16),
        (((1,), (0,)), ((), ())), preferred_element_type=jnp.float32)
    q = vt[0:1, :] * vt[0:1, :] + vt[1:2, :] * vt[1:2, :] + vt[2:3, :] * vt[2:3, :]
    qs = vs[:, 0:1] * vs[:, 0:1] + vs[:, 1:2] * vs[:, 1:2] + vs[:, 2:3] * vs[:, 2:3]
    d = inner * (-2.0) + q + qs

    col = lax.broadcasted_iota(jnp.int32, (PN, N), 1)
    col16 = lax.broadcasted_iota(jnp.int32, (PN, NEIGHBOR_NUM), 1)
    base = pl.program_id(0) * N
    out = jnp.zeros((PN, NEIGHBOR_NUM), jnp.int32)
    for t in range(K1):
        m = jnp.min(d, axis=1, keepdims=True)
        idx = jnp.min(jnp.where(d == m, col, N), axis=1, keepdims=True)
        if t > 0:
            out = jnp.where(col16 == (t - 1), idx + base, out)
        if t < K1 - 1:
            d = jnp.where(col == idx, jnp.inf, d)
    out_ref[0] = out


def _knn_indices(vs, vt):
    return pl.pallas_call(
        _knn_body,
        grid=(BS,),
        in_specs=[
            pl.BlockSpec((1, PN, 3), lambda b: (b, 0, 0)),
            pl.BlockSpec((1, 3, N), lambda b: (b, 0, 0)),
        ],
        out_specs=pl.BlockSpec((1, PN, NEIGHBOR_NUM), lambda b: (b, 0, 0)),
        out_shape=jax.ShapeDtypeStruct((BS, PN, NEIGHBOR_NUM), jnp.int32),
    )(vs, vt)


_ROWS = BS * PN
_NW = 32
_RPW = _ROWS // _NW
_L = 16


_RPC = 2
_GR = _RPC * NEIGHBOR_NUM
_NCHUNK = _RPW // _RPC


def _pool_sc(f_hbm, idx_hbm):
    mesh = plsc.VectorSubcoreMesh(core_axis_name="c", subcore_axis_name="s")

    @functools.partial(
        pl.kernel,
        mesh=mesh,
        out_type=jax.ShapeDtypeStruct((_ROWS, C), jnp.float32),
        scratch_types=[
            pltpu.VMEM((_RPW * NEIGHBOR_NUM,), jnp.int32),
            pltpu.VMEM((_GR, C), jnp.float32),
            pltpu.VMEM((_GR, C), jnp.float32),
            pltpu.VMEM((_RPW, C), jnp.float32),
            pltpu.SemaphoreType.DMA,
            pltpu.SemaphoreType.DMA,
        ],
    )
    def k(f_ref, idx_ref, out_ref, idx_v, g0, g1, out_v, sg0, sg1):
        wid = lax.axis_index("s") * 2 + lax.axis_index("c")
        base = wid * _RPW
        pltpu.sync_copy(idx_ref.at[pl.ds(base * NEIGHBOR_NUM, _RPW * NEIGHBOR_NUM)],
                        idx_v)

        def gather(c, buf, sem):
            pltpu.make_async_copy(
                f_ref.at[idx_v.at[pl.ds(c * _GR, _GR)]], buf, sem).start()

        def wait(buf, sem):
            pltpu.make_async_copy(
                f_ref.at[idx_v.at[pl.ds(0, _GR)]], buf, sem).wait()

        def compute(c, buf):
            def row(r, carry):
                for c0 in range(0, C, _L):
                    acc = buf[r * NEIGHBOR_NUM, pl.ds(c0, _L)]
                    for j in range(1, NEIGHBOR_NUM):
                        acc = jnp.maximum(acc, buf[r * NEIGHBOR_NUM + j, pl.ds(c0, _L)])
                    out_v[c * _RPC + r, pl.ds(c0, _L)] = acc
                return carry
            lax.fori_loop(0, _RPC, row, 0)

        gather(0, g0, sg0)

        def body(i, carry):
            c0 = i * 2
            c1 = c0 + 1
            gather(c1, g1, sg1)
            wait(g0, sg0)
            compute(c0, g0)

            @pl.when(c1 + 1 < _NCHUNK)
            def _():
                gather(c1 + 1, g0, sg0)

            wait(g1, sg1)
            compute(c1, g1)
            return carry

        lax.fori_loop(0, _NCHUNK // 2, body, 0)
        pltpu.sync_copy(out_v, out_ref.at[pl.ds(base, _RPW)])

    return k(f_hbm, idx_hbm)


def kernel(vertices, feature_map):
    sidx = jnp.asarray(_SAMPLE_IDX)
    vs = jnp.take(vertices, sidx, axis=1)
    vt = jnp.transpose(vertices, (0, 2, 1))
    nbr = _knn_indices(vs, vt)
    f_flat = feature_map.reshape(BS * N, C)
    idx_flat = nbr.reshape(_ROWS * NEIGHBOR_NUM)
    pooled = _pool_sc(f_flat, idx_flat)
    return vs, pooled.reshape(BS, PN, C)


# speedup vs baseline: 63.7029x; 1.4059x over previous
import functools

import jax
import jax.numpy as jnp
import numpy as np
from jax import lax
from jax.experimental import pallas as pl
from jax.experimental.pallas import tpu as pltpu
from jax.experimental.pallas import tpu_sc as plsc

POOLING_RATE = 4
NEIGHBOR_NUM = 16
BS = 8
N = 2048
C = 512
PN = N // POOLING_RATE
K1 = NEIGHBOR_NUM + 1

_SAMPLE_IDX = np.asarray(
    jax.random.permutation(jax.random.key(42), N)[:PN], dtype=np.int32)


def _knn_body(vs_ref, vt_ref, out_ref):
    vs = vs_ref[0]
    vt = vt_ref[0]
    inner = lax.dot_general(
        vs.astype(jnp.bfloat16), vt.astype(jnp.bfloat16),
        (((1,), (0,)), ((), ())), preferred_element_type=jnp.float32)
    q = vt[0:1, :] * vt[0:1, :] + vt[1:2, :] * vt[1:2, :] + vt[2:3, :] * vt[2:3, :]
    qs = vs[:, 0:1] * vs[:, 0:1] + vs[:, 1:2] * vs[:, 1:2] + vs[:, 2:3] * vs[:, 2:3]
    d = inner * (-2.0) + q + qs

    DEPTH = 4
    NG = N // 128
    INF = jnp.float32(jnp.inf)
    col128 = lax.broadcasted_iota(jnp.int32, (PN, 128), 1)
    col16 = lax.broadcasted_iota(jnp.int32, (PN, NEIGHBOR_NUM), 1)

    R = [jnp.full((PN, 128), INF, jnp.float32) for _ in range(DEPTH)]
    G = [jnp.zeros((PN, 128), jnp.int32) for _ in range(DEPTH)]
    for g in range(NG):
        x = d[:, g * 128:(g + 1) * 128]
        b = [x < R[k] for k in range(DEPTH)]
        for k in range(DEPTH - 1, 0, -1):
            R[k] = jnp.where(b[k], jnp.where(b[k - 1], R[k - 1], x), R[k])
            G[k] = jnp.where(b[k], jnp.where(b[k - 1], G[k - 1], g), G[k])
        R[0] = jnp.where(b[0], x, R[0])
        G[0] = jnp.where(b[0], g, G[0])

    base = pl.program_id(0) * N
    big = jnp.int32(1 << 30)
    out = jnp.zeros((PN, NEIGHBOR_NUM), jnp.int32)
    for t in range(K1):
        m = jnp.min(R[0], axis=1, keepdims=True)
        key = jnp.where(R[0] == m, G[0] * 128 + col128, big)
        idx = jnp.min(key, axis=1, keepdims=True)
        if t > 0:
            out = jnp.where(col16 == (t - 1), idx + base, out)
        if t < K1 - 1:
            lm = col128 == (idx & 127)
            for k in range(DEPTH - 1):
                R[k] = jnp.where(lm, R[k + 1], R[k])
                G[k] = jnp.where(lm, G[k + 1], G[k])
            R[DEPTH - 1] = jnp.where(lm, INF, R[DEPTH - 1])
    out_ref[0] = out


def _knn_indices(vs, vt):
    return pl.pallas_call(
        _knn_body,
        grid=(BS,),
        in_specs=[
            pl.BlockSpec((1, PN, 3), lambda b: (b, 0, 0)),
            pl.BlockSpec((1, 3, N), lambda b: (b, 0, 0)),
        ],
        out_specs=pl.BlockSpec((1, PN, NEIGHBOR_NUM), lambda b: (b, 0, 0)),
        out_shape=jax.ShapeDtypeStruct((BS, PN, NEIGHBOR_NUM), jnp.int32),
    )(vs, vt)


_ROWS = BS * PN
_NW = 32
_RPW = _ROWS // _NW
_L = 16


_RPC = 2
_GR = _RPC * NEIGHBOR_NUM
_NCHUNK = _RPW // _RPC


def _pool_sc(f_hbm, idx_hbm):
    mesh = plsc.VectorSubcoreMesh(core_axis_name="c", subcore_axis_name="s")

    @functools.partial(
        pl.kernel,
        mesh=mesh,
        out_type=jax.ShapeDtypeStruct((_ROWS, C), jnp.float32),
        scratch_types=[
            pltpu.VMEM((_RPW * NEIGHBOR_NUM,), jnp.int32),
            pltpu.VMEM((_GR, C), jnp.float32),
            pltpu.VMEM((_GR, C), jnp.float32),
            pltpu.VMEM((_RPW, C), jnp.float32),
            pltpu.SemaphoreType.DMA,
            pltpu.SemaphoreType.DMA,
        ],
    )
    def k(f_ref, idx_ref, out_ref, idx_v, g0, g1, out_v, sg0, sg1):
        wid = lax.axis_index("s") * 2 + lax.axis_index("c")
        base = wid * _RPW
        pltpu.sync_copy(idx_ref.at[pl.ds(base * NEIGHBOR_NUM, _RPW * NEIGHBOR_NUM)],
                        idx_v)

        def gather(c, buf, sem):
            pltpu.make_async_copy(
                f_ref.at[idx_v.at[pl.ds(c * _GR, _GR)]], buf, sem).start()

        def wait(buf, sem):
            pltpu.make_async_copy(
                f_ref.at[idx_v.at[pl.ds(0, _GR)]], buf, sem).wait()

        def compute(c, buf):
            def row(r, carry):
                for c0 in range(0, C, _L):
                    acc = buf[r * NEIGHBOR_NUM, pl.ds(c0, _L)]
                    for j in range(1, NEIGHBOR_NUM):
                        acc = jnp.maximum(acc, buf[r * NEIGHBOR_NUM + j, pl.ds(c0, _L)])
                    out_v[c * _RPC + r, pl.ds(c0, _L)] = acc
                return carry
            lax.fori_loop(0, _RPC, row, 0)

        gather(0, g0, sg0)

        def body(i, carry):
            c0 = i * 2
            c1 = c0 + 1
            gather(c1, g1, sg1)
            wait(g0, sg0)
            compute(c0, g0)

            @pl.when(c1 + 1 < _NCHUNK)
            def _():
                gather(c1 + 1, g0, sg0)

            wait(g1, sg1)
            compute(c1, g1)
            return carry

        lax.fori_loop(0, _NCHUNK // 2, body, 0)
        pltpu.sync_copy(out_v, out_ref.at[pl.ds(base, _RPW)])

    return k(f_hbm, idx_hbm)


def kernel(vertices, feature_map):
    sidx = jnp.asarray(_SAMPLE_IDX)
    vs = jnp.take(vertices, sidx, axis=1)
    vt = jnp.transpose(vertices, (0, 2, 1))
    nbr = _knn_indices(vs, vt)
    f_flat = feature_map.reshape(BS * N, C)
    idx_flat = nbr.reshape(_ROWS * NEIGHBOR_NUM)
    pooled = _pool_sc(f_flat, idx_flat)
    return vs, pooled.reshape(BS, PN, C)


# speedup vs baseline: 63.7996x; 1.0015x over previous
import functools

import jax
import jax.numpy as jnp
import numpy as np
from jax import lax
from jax.experimental import pallas as pl
from jax.experimental.pallas import tpu as pltpu
from jax.experimental.pallas import tpu_sc as plsc

POOLING_RATE = 4
NEIGHBOR_NUM = 16
BS = 8
N = 2048
C = 512
PN = N // POOLING_RATE
K1 = NEIGHBOR_NUM + 1

_SAMPLE_IDX = np.asarray(
    jax.random.permutation(jax.random.key(42), N)[:PN], dtype=np.int32)


def _knn_body(vs_ref, vt_ref, out_ref):
    vs = vs_ref[0]
    vt = vt_ref[0]
    inner = lax.dot_general(
        vs.astype(jnp.bfloat16), vt.astype(jnp.bfloat16),
        (((1,), (0,)), ((), ())), preferred_element_type=jnp.float32)
    q = vt[0:1, :] * vt[0:1, :] + vt[1:2, :] * vt[1:2, :] + vt[2:3, :] * vt[2:3, :]
    qs = vs[:, 0:1] * vs[:, 0:1] + vs[:, 1:2] * vs[:, 1:2] + vs[:, 2:3] * vs[:, 2:3]
    d = inner * (-2.0) + q + qs

    DEPTH = 4
    NG = N // 128
    INF = jnp.float32(jnp.inf)
    col128 = lax.broadcasted_iota(jnp.int32, (PN, 128), 1)
    col16 = lax.broadcasted_iota(jnp.int32, (PN, NEIGHBOR_NUM), 1)

    R = [jnp.full((PN, 128), INF, jnp.float32) for _ in range(DEPTH)]
    G = [jnp.zeros((PN, 128), jnp.int32) for _ in range(DEPTH)]
    for g in range(NG):
        x = d[:, g * 128:(g + 1) * 128]
        b = [x < R[k] for k in range(DEPTH)]
        for k in range(DEPTH - 1, 0, -1):
            R[k] = jnp.where(b[k], jnp.where(b[k - 1], R[k - 1], x), R[k])
            G[k] = jnp.where(b[k], jnp.where(b[k - 1], G[k - 1], g), G[k])
        R[0] = jnp.where(b[0], x, R[0])
        G[0] = jnp.where(b[0], g, G[0])

    base = pl.program_id(0) * N
    big = jnp.int32(1 << 30)
    out = jnp.zeros((PN, NEIGHBOR_NUM), jnp.int32)
    for t in range(K1):
        m = jnp.min(R[0], axis=1, keepdims=True)
        key = jnp.where(R[0] == m, G[0] * 128 + col128, big)
        idx = jnp.min(key, axis=1, keepdims=True)
        if t > 0:
            out = jnp.where(col16 == (t - 1), idx + base, out)
        if t < K1 - 1:
            lm = col128 == (idx & 127)
            for k in range(DEPTH - 1):
                R[k] = jnp.where(lm, R[k + 1], R[k])
                G[k] = jnp.where(lm, G[k + 1], G[k])
            R[DEPTH - 1] = jnp.where(lm, INF, R[DEPTH - 1])
    out_ref[0] = out


def _knn_indices(vs, vt):
    return pl.pallas_call(
        _knn_body,
        grid=(BS,),
        in_specs=[
            pl.BlockSpec((1, PN, 3), lambda b: (b, 0, 0)),
            pl.BlockSpec((1, 3, N), lambda b: (b, 0, 0)),
        ],
        out_specs=pl.BlockSpec((1, PN, NEIGHBOR_NUM), lambda b: (b, 0, 0)),
        out_shape=jax.ShapeDtypeStruct((BS, PN, NEIGHBOR_NUM), jnp.int32),
    )(vs, vt)


_ROWS = BS * PN
_NW = 32
_RPW = _ROWS // _NW
_L = 16


_RPC = 4
_GR = _RPC * NEIGHBOR_NUM
_NCHUNK = _RPW // _RPC


def _pool_sc(f_hbm, idx_hbm):
    mesh = plsc.VectorSubcoreMesh(core_axis_name="c", subcore_axis_name="s")

    @functools.partial(
        pl.kernel,
        mesh=mesh,
        out_type=jax.ShapeDtypeStruct((_ROWS, C), jnp.float32),
        scratch_types=[
            pltpu.VMEM((_RPW * NEIGHBOR_NUM,), jnp.int32),
            pltpu.VMEM((_GR, C), jnp.float32),
            pltpu.VMEM((_GR, C), jnp.float32),
            pltpu.VMEM((_RPC, C), jnp.float32),
            pltpu.VMEM((_RPC, C), jnp.float32),
            pltpu.SemaphoreType.DMA,
            pltpu.SemaphoreType.DMA,
            pltpu.SemaphoreType.DMA,
            pltpu.SemaphoreType.DMA,
        ],
    )
    def k(f_ref, idx_ref, out_ref, idx_v, g0, g1, o0, o1, sg0, sg1, so0, so1):
        wid = lax.axis_index("s") * 2 + lax.axis_index("c")
        base = wid * _RPW
        pltpu.sync_copy(idx_ref.at[pl.ds(base * NEIGHBOR_NUM, _RPW * NEIGHBOR_NUM)],
                        idx_v)

        def gather(c, buf, sem):
            pltpu.make_async_copy(
                f_ref.at[idx_v.at[pl.ds(c * _GR, _GR)]], buf, sem).start()

        def gwait(buf, sem):
            pltpu.make_async_copy(
                f_ref.at[idx_v.at[pl.ds(0, _GR)]], buf, sem).wait()

        def owait(obuf, sem):
            pltpu.make_async_copy(obuf, out_ref.at[pl.ds(0, _RPC)], sem).wait()

        def compute(buf, obuf):
            def row(r, carry):
                for c0 in range(0, C, _L):
                    acc = buf[r * NEIGHBOR_NUM, pl.ds(c0, _L)]
                    for j in range(1, NEIGHBOR_NUM):
                        acc = jnp.maximum(acc, buf[r * NEIGHBOR_NUM + j, pl.ds(c0, _L)])
                    obuf[r, pl.ds(c0, _L)] = acc
                return carry
            lax.fori_loop(0, _RPC, row, 0)

        def phase(c, gcur, scur, gnxt, snxt, ocur, socur):
            @pl.when(c + 1 < _NCHUNK)
            def _():
                gather(c + 1, gnxt, snxt)

            gwait(gcur, scur)

            @pl.when(c >= 2)
            def _():
                owait(ocur, socur)

            compute(gcur, ocur)
            pltpu.make_async_copy(
                ocur, out_ref.at[pl.ds(base + c * _RPC, _RPC)], socur).start()

        gather(0, g0, sg0)

        def body(i, carry):
            phase(i * 2, g0, sg0, g1, sg1, o0, so0)
            phase(i * 2 + 1, g1, sg1, g0, sg0, o1, so1)
            return carry

        lax.fori_loop(0, _NCHUNK // 2, body, 0)
        owait(o0, so0)
        owait(o1, so1)

    return k(f_hbm, idx_hbm)


def kernel(vertices, feature_map):
    sidx = jnp.asarray(_SAMPLE_IDX)
    vs = jnp.take(vertices, sidx, axis=1)
    vt = jnp.transpose(vertices, (0, 2, 1))
    nbr = _knn_indices(vs, vt)
    f_flat = feature_map.reshape(BS * N, C)
    idx_flat = nbr.reshape(_ROWS * NEIGHBOR_NUM)
    pooled = _pool_sc(f_flat, idx_flat)
    return vs, pooled.reshape(BS, PN, C)
